# baseline (speedup 1.0000x reference)
import functools
import math

import jax
import jax.numpy as jnp
import numpy as np
from jax import lax
from jax.experimental import pallas as pl
from jax.experimental.pallas import tpu as pltpu
from jax.experimental.pallas import tpu_sc as plsc

_CHUNK = 64


@functools.lru_cache(maxsize=None)
def _make_sc_gather(vocab: int, emb_dim: int, n_tok: int):
    info = plsc.get_sparse_core_info()
    nw = info.num_cores * info.num_subcores
    assert n_tok % (nw * _CHUNK) == 0
    chunks_per_w = n_tok // (nw * _CHUNK)

    mesh = plsc.VectorSubcoreMesh(core_axis_name="c", subcore_axis_name="s")

    @functools.partial(
        pl.kernel,
        out_type=jax.ShapeDtypeStruct((n_tok, emb_dim), jnp.float32),
        mesh=mesh,
        scratch_types=[
            pltpu.VMEM((_CHUNK,), jnp.int32),
            pltpu.VMEM((_CHUNK, emb_dim), jnp.float32),
            pltpu.SemaphoreType.DMA,
        ],
    )
    def gather_kernel(table_hbm, idx_hbm, out_hbm, idx_v, rows_v, sem):
        wid = lax.axis_index("s") * info.num_cores + lax.axis_index("c")

        def body(j, carry):
            base = (wid * chunks_per_w + j) * _CHUNK
            pltpu.sync_copy(idx_hbm.at[pl.ds(base, _CHUNK)], idx_v)
            pltpu.async_copy(table_hbm.at[idx_v], rows_v, sem).wait()
            pltpu.sync_copy(rows_v, out_hbm.at[pl.ds(base, _CHUNK)])
            return carry

        lax.fori_loop(0, chunks_per_w, body, 0)

    return gather_kernel


def _tc_body(x_ref, g_ref, pe_ref, w_ref, b_ref, o_ref, *, scale):
    mask = (x_ref[0] != 0).astype(jnp.float32)
    emb = g_ref[0] * (mask * scale)
    emb = emb + pe_ref[0]
    o_ref[0] = (
        jnp.dot(emb, w_ref[...], preferred_element_type=jnp.float32)
        + b_ref[...]
    )


def _positional_encoding(seq_len, d):
    position = jnp.arange(seq_len, dtype=jnp.float32)[:, None]
    div_term = jnp.exp(
        jnp.arange(0, d, 2, dtype=jnp.float32) * (-np.log(10000.0) / d)
    )
    pe = jnp.zeros((seq_len, d), dtype=jnp.float32)
    pe = pe.at[:, 0::2].set(jnp.sin(position * div_term))
    pe = pe.at[:, 1::2].set(jnp.cos(position * div_term))
    return pe


def kernel(x, x_pad_mask, emb_table, proj_w, proj_b):
    seq, batch = x.shape
    vocab, emb_dim = emb_table.shape
    d_model = proj_w.shape[1]
    n_tok = seq * batch
    scale = math.sqrt(float(emb_dim))

    x = x.astype(jnp.int32)
    gathered = _make_sc_gather(vocab, emb_dim, n_tok)(
        emb_table, x.reshape(n_tok)
    )

    pe = _positional_encoding(seq, emb_dim).reshape(seq, 1, emb_dim)
    x3 = x.reshape(seq, batch, 1)
    g3 = gathered.reshape(seq, batch, emb_dim)
    b2 = proj_b.reshape(1, d_model)

    out = pl.pallas_call(
        functools.partial(_tc_body, scale=scale),
        grid=(seq,),
        in_specs=[
            pl.BlockSpec((1, batch, 1), lambda s: (s, 0, 0)),
            pl.BlockSpec((1, batch, emb_dim), lambda s: (s, 0, 0)),
            pl.BlockSpec((1, 1, emb_dim), lambda s: (s, 0, 0)),
            pl.BlockSpec((emb_dim, d_model), lambda s: (0, 0)),
            pl.BlockSpec((1, d_model), lambda s: (0, 0)),
        ],
        out_specs=pl.BlockSpec((1, batch, d_model), lambda s: (s, 0, 0)),
        out_shape=jax.ShapeDtypeStruct((seq, batch, d_model), jnp.float32),
    )(x3, g3, pe, proj_w, b2)
    return out


# speedup vs baseline: 1.0973x; 1.0973x over previous
import functools
import math

import jax
import jax.numpy as jnp
import numpy as np
from jax import lax
from jax.experimental import pallas as pl
from jax.experimental.pallas import tpu as pltpu
from jax.experimental.pallas import tpu_sc as plsc

_CHUNK = 64


@functools.lru_cache(maxsize=None)
def _make_sc_gather(vocab: int, emb_dim: int, n_tok: int):
    info = plsc.get_sparse_core_info()
    nw = info.num_cores * info.num_subcores
    assert n_tok % (nw * _CHUNK) == 0
    chunks_per_w = n_tok // (nw * _CHUNK)

    mesh = plsc.VectorSubcoreMesh(core_axis_name="c", subcore_axis_name="s")

    @functools.partial(
        pl.kernel,
        out_type=jax.ShapeDtypeStruct((n_tok, emb_dim), jnp.float32),
        mesh=mesh,
        scratch_types=[
            pltpu.VMEM((_CHUNK,), jnp.int32),
            pltpu.VMEM((_CHUNK, emb_dim), jnp.float32),
            pltpu.SemaphoreType.DMA,
        ],
    )
    def gather_kernel(table_hbm, idx_hbm, out_hbm, idx_v, rows_v, sem):
        wid = lax.axis_index("s") * info.num_cores + lax.axis_index("c")

        def body(j, carry):
            base = (wid * chunks_per_w + j) * _CHUNK
            pltpu.sync_copy(idx_hbm.at[pl.ds(base, _CHUNK)], idx_v)
            pltpu.async_copy(table_hbm.at[idx_v], rows_v, sem).wait()
            pltpu.sync_copy(rows_v, out_hbm.at[pl.ds(base, _CHUNK)])
            return carry

        lax.fori_loop(0, chunks_per_w, body, 0)

    return gather_kernel


def _tc_body(x_ref, g_ref, pe_ref, w_ref, b_ref, o_ref, *, scale):
    mask = (x_ref[0] != 0).astype(jnp.float32)
    emb = g_ref[0] * (mask * scale)
    emb = emb + pe_ref[0]
    o_ref[0] = (
        jnp.dot(emb, w_ref[...], preferred_element_type=jnp.float32)
        + b_ref[...]
    )


def _positional_encoding(seq_len, d):
    position = jnp.arange(seq_len, dtype=jnp.float32)[:, None]
    div_term = jnp.exp(
        jnp.arange(0, d, 2, dtype=jnp.float32) * (-np.log(10000.0) / d)
    )
    pe = jnp.zeros((seq_len, d), dtype=jnp.float32)
    pe = pe.at[:, 0::2].set(jnp.sin(position * div_term))
    pe = pe.at[:, 1::2].set(jnp.cos(position * div_term))
    return pe


def _tc_body_acc(x_ref, g_ref, pe_ref, w_ref, b_ref, prev_ref, o_ref, *,
                 scale):
    del prev_ref
    _tc_body(x_ref, g_ref, pe_ref, w_ref, b_ref, o_ref, scale=scale)


_N_SLICES = 5


def kernel(x, x_pad_mask, emb_table, proj_w, proj_b):
    seq, batch = x.shape
    vocab, emb_dim = emb_table.shape
    d_model = proj_w.shape[1]
    scale = math.sqrt(float(emb_dim))

    x = x.astype(jnp.int32)
    idx = x.reshape(seq * batch)

    s_per = seq // _N_SLICES
    tok_per = s_per * batch
    sc_gather = _make_sc_gather(vocab, emb_dim, tok_per)
    gathered = [
        sc_gather(emb_table, idx[k * tok_per:(k + 1) * tok_per])
        for k in range(_N_SLICES)
    ]

    pe = _positional_encoding(seq, emb_dim).reshape(seq, 1, emb_dim)
    x3 = x.reshape(seq, batch, 1)
    b2 = proj_b.reshape(1, d_model)

    out_shape = jax.ShapeDtypeStruct((seq, batch, d_model), jnp.float32)
    out = None
    for k in range(_N_SLICES):
        off = k * s_per
        g3 = gathered[k].reshape(s_per, batch, emb_dim)
        in_specs = [
            pl.BlockSpec((1, batch, 1), lambda s, off=off: (off + s, 0, 0)),
            pl.BlockSpec((1, batch, emb_dim), lambda s: (s, 0, 0)),
            pl.BlockSpec((1, 1, emb_dim), lambda s, off=off: (off + s, 0, 0)),
            pl.BlockSpec((emb_dim, d_model), lambda s: (0, 0)),
            pl.BlockSpec((1, d_model), lambda s: (0, 0)),
        ]
        out_spec = pl.BlockSpec(
            (1, batch, d_model), lambda s, off=off: (off + s, 0, 0)
        )
        if out is None:
            out = pl.pallas_call(
                functools.partial(_tc_body, scale=scale),
                grid=(s_per,),
                in_specs=in_specs,
                out_specs=out_spec,
                out_shape=out_shape,
            )(x3, g3, pe, proj_w, b2)
        else:
            prev_spec = pl.BlockSpec(
                (1, 8, 128), lambda s, off=off: (off + s, 0, 0)
            )
            out = pl.pallas_call(
                functools.partial(_tc_body_acc, scale=scale),
                grid=(s_per,),
                in_specs=in_specs + [prev_spec],
                out_specs=out_spec,
                out_shape=out_shape,
                input_output_aliases={5: 0},
            )(x3, g3, pe, proj_w, b2, out)
    return out


# speedup vs baseline: 1.2171x; 1.1092x over previous
import functools
import math

import jax
import jax.numpy as jnp
import numpy as np
from jax import lax
from jax.experimental import pallas as pl
from jax.experimental.pallas import tpu as pltpu
from jax.experimental.pallas import tpu_sc as plsc

_CHUNK = 64


@functools.lru_cache(maxsize=None)
def _make_sc_gather(vocab: int, emb_dim: int, n_tok: int):
    info = plsc.get_sparse_core_info()
    nw = info.num_cores * info.num_subcores
    assert n_tok % (nw * _CHUNK) == 0
    chunks_per_w = n_tok // (nw * _CHUNK)

    mesh = plsc.VectorSubcoreMesh(core_axis_name="c", subcore_axis_name="s")

    @functools.partial(
        pl.kernel,
        out_type=jax.ShapeDtypeStruct((n_tok, emb_dim), jnp.float32),
        mesh=mesh,
        scratch_types=[
            pltpu.VMEM((_CHUNK,), jnp.int32),
            pltpu.VMEM((_CHUNK, emb_dim), jnp.float32),
            pltpu.SemaphoreType.DMA,
        ],
    )
    def gather_kernel(table_hbm, idx_hbm, out_hbm, idx_v, rows_v, sem):
        wid = lax.axis_index("s") * info.num_cores + lax.axis_index("c")

        def body(j, carry):
            base = (wid * chunks_per_w + j) * _CHUNK
            pltpu.sync_copy(idx_hbm.at[pl.ds(base, _CHUNK)], idx_v)
            pltpu.async_copy(table_hbm.at[idx_v], rows_v, sem).wait()
            pltpu.sync_copy(rows_v, out_hbm.at[pl.ds(base, _CHUNK)])
            return carry

        lax.fori_loop(0, chunks_per_w, body, 0)

    return gather_kernel


def _tc_body(x_ref, g_ref, pe_ref, w_ref, b_ref, r0_ref, o_ref, *, scale):
    b_dim, e_dim = g_ref.shape[1], g_ref.shape[2]
    z = (x_ref[0] == 0).astype(jnp.float32)
    z3 = lax.broadcast_in_dim(z, (8, b_dim // 8, e_dim), (0, 1))
    r3 = lax.broadcast_in_dim(r0_ref[0], (8, b_dim // 8, e_dim), (2,))
    g3 = g_ref[0].reshape(8, b_dim // 8, e_dim)
    emb = (g3 - z3 * r3).reshape(b_dim, e_dim) * scale + pe_ref[0]
    o_ref[0] = (
        jnp.dot(emb.astype(jnp.bfloat16), w_ref[...],
                preferred_element_type=jnp.float32)
        + b_ref[...]
    )


def _positional_encoding(seq_len, d):
    position = jnp.arange(seq_len, dtype=jnp.float32)[:, None]
    div_term = jnp.exp(
        jnp.arange(0, d, 2, dtype=jnp.float32) * (-np.log(10000.0) / d)
    )
    pe = jnp.zeros((seq_len, d), dtype=jnp.float32)
    pe = pe.at[:, 0::2].set(jnp.sin(position * div_term))
    pe = pe.at[:, 1::2].set(jnp.cos(position * div_term))
    return pe


def _tc_body_acc(x_ref, g_ref, pe_ref, w_ref, b_ref, r0_ref, prev_ref,
                 o_ref, *, scale):
    del prev_ref
    _tc_body(x_ref, g_ref, pe_ref, w_ref, b_ref, r0_ref, o_ref, scale=scale)


_N_SLICES = 5


def kernel(x, x_pad_mask, emb_table, proj_w, proj_b):
    seq, batch = x.shape
    vocab, emb_dim = emb_table.shape
    d_model = proj_w.shape[1]
    scale = math.sqrt(float(emb_dim))

    x = x.astype(jnp.int32)
    idx = x.reshape(seq * batch)

    s_per = seq // _N_SLICES
    tok_per = s_per * batch
    sc_gather = _make_sc_gather(vocab, emb_dim, tok_per)
    gathered = [
        sc_gather(emb_table, idx[k * tok_per:(k + 1) * tok_per])
        for k in range(_N_SLICES)
    ]

    pe = _positional_encoding(seq, emb_dim).reshape(seq, 1, emb_dim)
    x3 = x.reshape(seq, batch // 128, 128)
    b2 = proj_b.reshape(1, d_model)
    wb = proj_w.astype(jnp.bfloat16)
    r0 = lax.slice(emb_table, (0, 0), (1, emb_dim))

    out_shape = jax.ShapeDtypeStruct((seq, batch, d_model), jnp.float32)
    out = None
    for k in range(_N_SLICES):
        off = k * s_per
        g3 = gathered[k].reshape(s_per, batch, emb_dim)
        in_specs = [
            pl.BlockSpec((1, batch // 128, 128),
                         lambda s, off=off: (off + s, 0, 0)),
            pl.BlockSpec((1, batch, emb_dim), lambda s: (s, 0, 0)),
            pl.BlockSpec((1, 1, emb_dim), lambda s, off=off: (off + s, 0, 0)),
            pl.BlockSpec((emb_dim, d_model), lambda s: (0, 0)),
            pl.BlockSpec((1, d_model), lambda s: (0, 0)),
            pl.BlockSpec((1, emb_dim), lambda s: (0, 0)),
        ]
        out_spec = pl.BlockSpec(
            (1, batch, d_model), lambda s, off=off: (off + s, 0, 0)
        )
        if out is None:
            out = pl.pallas_call(
                functools.partial(_tc_body, scale=scale),
                grid=(s_per,),
                in_specs=in_specs,
                out_specs=out_spec,
                out_shape=out_shape,
            )(x3, g3, pe, wb, b2, r0)
        else:
            prev_spec = pl.BlockSpec(
                (1, 8, 128), lambda s, off=off: (off + s, 0, 0)
            )
            out = pl.pallas_call(
                functools.partial(_tc_body_acc, scale=scale),
                grid=(s_per,),
                in_specs=in_specs + [prev_spec],
                out_specs=out_spec,
                out_shape=out_shape,
                input_output_aliases={6: 0},
            )(x3, g3, pe, wb, b2, r0, out)
    return out
